# baseline (speedup 1.0000x reference)
import jax
import jax.numpy as jnp
import numpy as np
from jax.experimental import pallas as pl
from jax.experimental.pallas import tpu as pltpu

_NCORES = 2
_NBUF = 4
_GMAX = 32


def _chunk_plan(base, nrows):
    ramp = [4, 8, 16]
    tail = [8, 4]
    bulk_rows = nrows - sum(ramp) - sum(tail)
    sizes = list(ramp)
    sizes += [_GMAX] * (bulk_rows // _GMAX)
    if bulk_rows % _GMAX:
        sizes.append(bulk_rows % _GMAX)
    sizes += tail
    starts = (base + np.cumsum([0] + sizes[:-1])).tolist()
    return list(zip(starts, sizes))


def _run_plan(chunks, slow_map, src, fast, slow, bufs, in_sems, out_sems):
    nchunks = len(chunks)

    def in_copy(g):
        b = g % _NBUF
        s0, sz = chunks[g]
        return pltpu.make_async_copy(
            src.at[pl.ds(s0, sz)], bufs.at[b].at[pl.ds(0, sz)],
            in_sems.at[b])

    def out_copies(g):
        b = g % _NBUF
        s0, sz = chunks[g]
        cps = [pltpu.make_async_copy(
            bufs.at[b].at[pl.ds(0, sz)], fast.at[pl.ds(s0, sz)],
            out_sems.at[b])]
        for k, r in slow_map[g]:
            cps.append(pltpu.make_async_copy(
                bufs.at[b].at[r], slow.at[k], out_sems.at[b]))
        return cps

    lag = _NBUF - 1
    for g in range(min(_NBUF, nchunks)):
        in_copy(g).start()
    for g in range(nchunks):
        in_copy(g).wait()
        for cp in out_copies(g):
            cp.start()
        gl = g - lag
        if gl >= 0:
            for cp in out_copies(gl):
                cp.wait()
            if gl + _NBUF < nchunks:
                in_copy(gl + _NBUF).start()
    for g in range(max(0, nchunks - lag), nchunks):
        for cp in out_copies(g):
            cp.wait()


def _make_body(plans):
    def body(src, fast, slow, bufs, in_sems, out_sems):
        pid = pl.program_id(0)
        for core, (chunks, slow_map) in enumerate(plans):
            @pl.when(pid == core)
            def _(chunks=chunks, slow_map=slow_map):
                _run_plan(chunks, slow_map, src, fast, slow,
                          bufs, in_sems, out_sems)

    return body


def _linspace_idx(stop, num):
    div = num - 1
    step = np.arange(div, dtype=np.float32) / np.float32(div)
    out = (np.float32(0.0) * (np.float32(1.0) - step)
           + np.float32(stop) * step)
    out = np.concatenate([out, np.array([stop], dtype=np.float32)])
    return out.astype(np.int32)


def kernel(frames):
    C, T, H, W = frames.shape
    alpha = 4
    n = T // alpha
    idx = _linspace_idx(float(T - 1), n)

    nrows = C * T
    per = nrows // _NCORES
    slow_rows = [(c * n + j, c * T + int(t))
                 for c in range(C) for j, t in enumerate(idx.tolist())]

    plans = []
    for core in range(_NCORES):
        chunks = _chunk_plan(core * per, per)
        slow_map = {g: [] for g in range(len(chunks))}
        for k, r in slow_rows:
            for g, (s0, sz) in enumerate(chunks):
                if s0 <= r < s0 + sz:
                    slow_map[g].append((k, r - s0))
                    break
        plans.append((chunks, slow_map))

    flat = frames.reshape(nrows, H, W)
    hbm = pl.BlockSpec(memory_space=pltpu.MemorySpace.HBM)
    fast_flat, slow_flat = pl.pallas_call(
        _make_body(plans),
        grid=(_NCORES,),
        in_specs=[hbm],
        out_specs=[hbm, hbm],
        out_shape=[
            jax.ShapeDtypeStruct((nrows, H, W), jnp.float32),
            jax.ShapeDtypeStruct((C * n, H, W), jnp.float32),
        ],
        scratch_shapes=[
            pltpu.VMEM((_NBUF, _GMAX, H, W), jnp.float32),
            pltpu.SemaphoreType.DMA((_NBUF,)),
            pltpu.SemaphoreType.DMA((_NBUF,)),
        ],
        compiler_params=pltpu.CompilerParams(
            dimension_semantics=("parallel",)),
    )(flat)
    return (slow_flat.reshape(C, n, H, W), fast_flat.reshape(C, T, H, W))


# speedup vs baseline: 1.2694x; 1.2694x over previous
import jax
import jax.numpy as jnp
import numpy as np
from jax.experimental import pallas as pl
from jax.experimental.pallas import tpu as pltpu

_G = 32
_NBUF = 6
_LAG = 3


def _make_body(nrows, slow_map):
    nchunks = nrows // _G

    def body(src, fast, slow, bufs, in_sems, out_sems):
        def in_copy(g):
            b = g % _NBUF
            return pltpu.make_async_copy(
                src.at[pl.ds(g * _G, _G)], bufs.at[b], in_sems.at[b])

        def out_copies(g):
            b = g % _NBUF
            cps = [pltpu.make_async_copy(
                bufs.at[b], fast.at[pl.ds(g * _G, _G)], out_sems.at[b])]
            for k, r in slow_map[g]:
                cps.append(pltpu.make_async_copy(
                    bufs.at[b].at[r], slow.at[k], out_sems.at[b]))
            return cps

        for g in range(min(_NBUF, nchunks)):
            in_copy(g).start()
        for g in range(nchunks):
            in_copy(g).wait()
            for cp in out_copies(g):
                cp.start()
            gl = g - _LAG
            if gl >= 0:
                for cp in out_copies(gl):
                    cp.wait()
                if gl + _NBUF < nchunks:
                    in_copy(gl + _NBUF).start()
        for g in range(max(0, nchunks - _LAG), nchunks):
            for cp in out_copies(g):
                cp.wait()

    return body


def _linspace_idx(stop, num):
    div = num - 1
    step = np.arange(div, dtype=np.float32) / np.float32(div)
    out = (np.float32(0.0) * (np.float32(1.0) - step)
           + np.float32(stop) * step)
    out = np.concatenate([out, np.array([stop], dtype=np.float32)])
    return out.astype(np.int32)


def kernel(frames):
    C, T, H, W = frames.shape
    alpha = 4
    n = T // alpha
    idx = _linspace_idx(float(T - 1), n)

    nrows = C * T
    slow_map = {g: [] for g in range(nrows // _G)}
    for c in range(C):
        for j, t in enumerate(idx.tolist()):
            r = c * T + int(t)
            slow_map[r // _G].append((c * n + j, r % _G))

    flat = frames.reshape(nrows, H, W)
    hbm = pl.BlockSpec(memory_space=pltpu.MemorySpace.HBM)
    fast_flat, slow_flat = pl.pallas_call(
        _make_body(nrows, slow_map),
        in_specs=[hbm],
        out_specs=[hbm, hbm],
        out_shape=[
            jax.ShapeDtypeStruct((nrows, H, W), jnp.float32),
            jax.ShapeDtypeStruct((C * n, H, W), jnp.float32),
        ],
        scratch_shapes=[
            pltpu.VMEM((_NBUF, _G, H, W), jnp.float32),
            pltpu.SemaphoreType.DMA((_NBUF,)),
            pltpu.SemaphoreType.DMA((_NBUF,)),
        ],
    )(flat)
    return (slow_flat.reshape(C, n, H, W), fast_flat.reshape(C, T, H, W))
